# baseline (speedup 1.0000x reference)
import functools

import jax
import jax.numpy as jnp
from jax import lax
from jax.experimental import pallas as pl
from jax.experimental.pallas import tpu as pltpu
from jax.experimental.pallas import tpu_sc as plsc

VOCAB = 1000000
EMB = 64
UNITS = 128
BATCH = 1024
SEQ = 50

_VB = 65536
_QB = _VB // 4
_NPB = (VOCAB + _VB - 1) // _VB
_QROWS = _NPB * _QB

_NW = 32
_ROWS = BATCH * SEQ
_RPW = _ROWS // _NW
_CH = 80
_NCH = _RPW // _CH

_BB = 512

_HI_MASK = -65536


def _pack_body(tT_ref, out_ref):
    x = lax.bitcast_convert_type(tT_ref[...], jnp.uint32)
    mask = jnp.uint32(0xffff0000)
    word_l = (x[:, :_QB] >> 16) | (x[:, 2 * _QB:3 * _QB] & mask)
    word_r = (x[:, _QB:2 * _QB] >> 16) | (x[:, 3 * _QB:] & mask)
    out_ref[:, :EMB] = lax.bitcast_convert_type(
        jnp.transpose(word_l), jnp.int32)
    out_ref[:, EMB:] = lax.bitcast_convert_type(
        jnp.transpose(word_r), jnp.int32)


def _tc_pack(tT):
    return pl.pallas_call(
        _pack_body,
        grid=(_NPB,),
        in_specs=[pl.BlockSpec((EMB, _VB), lambda i: (0, i))],
        out_specs=pl.BlockSpec((_QB, 128), lambda i: (i, 0)),
        out_shape=jax.ShapeDtypeStruct((_QROWS, 128), jnp.int32),
    )(tT)


def _sc_gather(table2, idx3):
    mesh = plsc.VectorSubcoreMesh(core_axis_name="c", subcore_axis_name="s")

    @functools.partial(
        pl.kernel,
        mesh=mesh,
        out_type=jax.ShapeDtypeStruct((_ROWS, 128), jnp.int32),
        scratch_types=[
            pltpu.VMEM((_NCH, _CH), jnp.int32),
            pltpu.VMEM((2, _CH, 128), jnp.int32),
            pltpu.SemaphoreType.DMA,
            pltpu.SemaphoreType.DMA,
        ],
    )
    def gather_kernel(table_hbm, idx_hbm, out_hbm, idx_v, buf_v, semg, semo):
        wid = lax.axis_index("s") * 2 + lax.axis_index("c")
        pltpu.sync_copy(idx_hbm.at[wid], idx_v)
        outs = []
        for j in range(_NCH):
            b = j % 2
            if j >= 2:
                outs[j - 2].wait()
            pltpu.async_copy(table_hbm.at[idx_v.at[j]], buf_v.at[b], semg).wait()
            outs.append(
                pltpu.async_copy(
                    buf_v.at[b],
                    out_hbm.at[pl.ds(wid * _RPW + j * _CH, _CH)],
                    semo,
                )
            )
        outs[-2].wait()
        outs[-1].wait()

    return gather_kernel(table2, idx3)


def _gru_body(xe_ref, par_ref, h0_ref, w2_ref, rw_ref, bi_ref, br_ref,
              out_ref, st_ref):
    h = h0_ref[...]
    w2 = w2_ref[...]
    rw = rw_ref[...]
    bi = bi_ref[...]
    br = br_ref[...]
    lane = lax.broadcasted_iota(jnp.int32, (_BB, 128), 1)
    hl = (lane >= EMB).astype(jnp.float32)
    for t in range(SEQ):
        w = xe_ref[t]
        pf = par_ref[t].reshape(_BB, 1)
        s_sel = jnp.where(pf >= 2.0, 1.0, 0.0)
        h_sel = pf - 2.0 * s_sel
        shamt = (16.0 * (1.0 - s_sel)).astype(jnp.int32)
        bits = jnp.left_shift(w, shamt) & _HI_MASK
        xt = lax.bitcast_convert_type(bits, jnp.float32)
        hmatch = jnp.where(hl == h_sel, 1.0, 0.0)
        xt_m = hmatch * xt
        gx = jnp.dot(xt_m, w2, preferred_element_type=jnp.float32) + bi
        gh = jnp.dot(h, rw, preferred_element_type=jnp.float32) + br
        xz = gx[:, :UNITS]
        xr = gx[:, UNITS:2 * UNITS]
        xh = gx[:, 2 * UNITS:]
        hz = gh[:, :UNITS]
        hr = gh[:, UNITS:2 * UNITS]
        hh = gh[:, 2 * UNITS:]
        z = jax.nn.sigmoid(xz + hz)
        r = jax.nn.sigmoid(xr + hr)
        hcand = jnp.tanh(xh + r * hh)
        h = z * h + (1.0 - z) * hcand
        out_ref[t] = h
    st_ref[...] = h


def _tc_gru(xe, par, hidden, w2, rw, bi, br):
    grid = (BATCH // _BB,)
    out, state = pl.pallas_call(
        _gru_body,
        grid=grid,
        in_specs=[
            pl.BlockSpec((SEQ, _BB, 128), lambda i: (0, i, 0)),
            pl.BlockSpec((SEQ, _BB), lambda i: (0, i)),
            pl.BlockSpec((_BB, UNITS), lambda i: (i, 0)),
            pl.BlockSpec((128, 3 * UNITS), lambda i: (0, 0)),
            pl.BlockSpec((UNITS, 3 * UNITS), lambda i: (0, 0)),
            pl.BlockSpec((1, 3 * UNITS), lambda i: (0, 0)),
            pl.BlockSpec((1, 3 * UNITS), lambda i: (0, 0)),
        ],
        out_specs=[
            pl.BlockSpec((SEQ, _BB, UNITS), lambda i: (0, i, 0)),
            pl.BlockSpec((_BB, UNITS), lambda i: (i, 0)),
        ],
        out_shape=[
            jax.ShapeDtypeStruct((SEQ, BATCH, UNITS), jnp.float32),
            jax.ShapeDtypeStruct((BATCH, UNITS), jnp.float32),
        ],
    )(xe, par, hidden, w2, rw, bi, br)
    return out, state


def kernel(x, hidden, emb_table, kernel, rec_kernel, bias_in, bias_rec):
    xi = x.astype(jnp.int32)
    blk = xi // _VB
    r = xi % _VB
    quarter = r // _QB
    qrow = blk * _QB + (r % _QB)
    idx = jnp.transpose(qrow).reshape(_NW, _NCH, _CH)
    par = jnp.transpose(quarter).astype(jnp.float32)
    table2 = _tc_pack(jnp.transpose(emb_table))
    rows = _sc_gather(table2, idx)
    xe = rows.reshape(SEQ, BATCH, 128)
    w2 = jnp.concatenate([kernel, kernel], axis=0)
    bi = bias_in.reshape(1, 3 * UNITS)
    br = bias_rec.reshape(1, 3 * UNITS)
    out, state = _tc_gru(xe, par, hidden, w2, rw=rec_kernel, bi=bi, br=br)
    return (jnp.swapaxes(out, 0, 1), state)


# speedup vs baseline: 1.0669x; 1.0669x over previous
import functools

import jax
import jax.numpy as jnp
from jax import lax
from jax.experimental import pallas as pl
from jax.experimental.pallas import tpu as pltpu
from jax.experimental.pallas import tpu_sc as plsc

VOCAB = 1000000
EMB = 64
UNITS = 128
BATCH = 1024
SEQ = 50

_VB = 32768
_QB = _VB // 4
_NPB = (VOCAB + _VB - 1) // _VB
_QROWS = _NPB * _QB

_NW = 32
_TH = SEQ // 2
_HROWS = BATCH * _TH
_RPW = _HROWS // _NW
_CH = 80
_NCH = _RPW // _CH

_BB = 512

_HI_MASK = -65536


def _pack_body(tT_ref, out_ref):
    x = lax.bitcast_convert_type(tT_ref[...], jnp.uint32)
    mask = jnp.uint32(0xffff0000)
    word_l = (x[:, :_QB] >> 16) | (x[:, 2 * _QB:3 * _QB] & mask)
    word_r = (x[:, _QB:2 * _QB] >> 16) | (x[:, 3 * _QB:] & mask)
    out_ref[:, :EMB] = lax.bitcast_convert_type(
        jnp.transpose(word_l), jnp.int32)
    out_ref[:, EMB:] = lax.bitcast_convert_type(
        jnp.transpose(word_r), jnp.int32)


def _tc_pack(tT):
    return pl.pallas_call(
        _pack_body,
        grid=(_NPB,),
        in_specs=[pl.BlockSpec((EMB, _VB), lambda i: (0, i))],
        out_specs=pl.BlockSpec((_QB, 128), lambda i: (i, 0)),
        out_shape=jax.ShapeDtypeStruct((_QROWS, 128), jnp.int32),
    )(tT)


def _sc_gather(table2, idx3):
    mesh = plsc.VectorSubcoreMesh(core_axis_name="c", subcore_axis_name="s")

    @functools.partial(
        pl.kernel,
        mesh=mesh,
        out_type=jax.ShapeDtypeStruct((_HROWS, 128), jnp.int32),
        scratch_types=[
            pltpu.VMEM((_NCH, _CH), jnp.int32),
            pltpu.VMEM((2, _CH, 128), jnp.int32),
            pltpu.SemaphoreType.DMA,
            pltpu.SemaphoreType.DMA,
        ],
    )
    def gather_kernel(table_hbm, idx_hbm, out_hbm, idx_v, buf_v, semg, semo):
        wid = lax.axis_index("s") * 2 + lax.axis_index("c")
        pltpu.sync_copy(idx_hbm.at[wid], idx_v)
        outs = []
        for j in range(_NCH):
            b = j % 2
            if j >= 2:
                outs[j - 2].wait()
            pltpu.async_copy(table_hbm.at[idx_v.at[j]], buf_v.at[b], semg).wait()
            outs.append(
                pltpu.async_copy(
                    buf_v.at[b],
                    out_hbm.at[pl.ds(wid * _RPW + j * _CH, _CH)],
                    semo,
                )
            )
        outs[-2].wait()
        outs[-1].wait()

    return gather_kernel(table2, idx3)


def _gru_body(xe_ref, par_ref, h0_ref, w2_ref, rw_ref, bi_ref, br_ref,
              out_ref, st_ref):
    h = h0_ref[...]
    w2 = w2_ref[...]
    rw = rw_ref[...]
    bi = bi_ref[...]
    br = br_ref[...]
    lane = lax.broadcasted_iota(jnp.int32, (_BB, 128), 1)
    hl = (lane >= EMB).astype(jnp.float32)
    for t in range(_TH):
        w = xe_ref[t]
        pf = par_ref[t].reshape(_BB, 1)
        s_sel = jnp.where(pf >= 2.0, 1.0, 0.0)
        h_sel = pf - 2.0 * s_sel
        shamt = (16.0 * (1.0 - s_sel)).astype(jnp.int32)
        bits = jnp.left_shift(w, shamt) & _HI_MASK
        xt = lax.bitcast_convert_type(bits, jnp.float32)
        hmatch = jnp.where(hl == h_sel, 1.0, 0.0)
        xt_m = hmatch * xt
        gx = jnp.dot(xt_m, w2, preferred_element_type=jnp.float32) + bi
        gh = jnp.dot(h, rw, preferred_element_type=jnp.float32) + br
        xz = gx[:, :UNITS]
        xr = gx[:, UNITS:2 * UNITS]
        xh = gx[:, 2 * UNITS:]
        hz = gh[:, :UNITS]
        hr = gh[:, UNITS:2 * UNITS]
        hh = gh[:, 2 * UNITS:]
        z = jax.nn.sigmoid(xz + hz)
        r = jax.nn.sigmoid(xr + hr)
        hcand = jnp.tanh(xh + r * hh)
        h = z * h + (1.0 - z) * hcand
        out_ref[t] = h
    st_ref[...] = h


def _gru_body_aliased(xe_ref, par_ref, h0_ref, w2_ref, rw_ref, bi_ref, br_ref,
                      obuf_ref, out_ref, st_ref):
    del obuf_ref
    _gru_body(xe_ref, par_ref, h0_ref, w2_ref, rw_ref, bi_ref, br_ref,
              out_ref, st_ref)


def _tc_gru_half(xe, par, h0, w2, rw, bi, br, half, obuf=None):
    grid = (BATCH // _BB,)
    in_specs = [
        pl.BlockSpec((_TH, _BB, 128), lambda i: (0, i, 0)),
        pl.BlockSpec((_TH, _BB), lambda i: (0, i)),
        pl.BlockSpec((_BB, UNITS), lambda i: (i, 0)),
        pl.BlockSpec((128, 3 * UNITS), lambda i: (0, 0)),
        pl.BlockSpec((UNITS, 3 * UNITS), lambda i: (0, 0)),
        pl.BlockSpec((1, 3 * UNITS), lambda i: (0, 0)),
        pl.BlockSpec((1, 3 * UNITS), lambda i: (0, 0)),
    ]
    args = [xe, par, h0, w2, rw, bi, br]
    body = _gru_body
    aliases = {}
    if obuf is not None:
        in_specs.append(pl.BlockSpec(memory_space=pl.ANY))
        args.append(obuf)
        body = _gru_body_aliased
        aliases = {7: 0}
    out, state = pl.pallas_call(
        body,
        grid=grid,
        in_specs=in_specs,
        out_specs=[
            pl.BlockSpec((_TH, _BB, UNITS), lambda i, h=half: (h, i, 0)),
            pl.BlockSpec((_BB, UNITS), lambda i: (i, 0)),
        ],
        out_shape=[
            jax.ShapeDtypeStruct((SEQ, BATCH, UNITS), jnp.float32),
            jax.ShapeDtypeStruct((BATCH, UNITS), jnp.float32),
        ],
        input_output_aliases=aliases,
    )(*args)
    return out, state


def kernel(x, hidden, emb_table, kernel, rec_kernel, bias_in, bias_rec):
    xi = x.astype(jnp.int32)
    blk = xi // _VB
    r = xi % _VB
    quarter = r // _QB
    qrow = blk * _QB + (r % _QB)
    qrow_t = jnp.transpose(qrow)
    par_t = jnp.transpose(quarter).astype(jnp.float32)
    table2 = _tc_pack(jnp.transpose(emb_table))
    w2 = jnp.concatenate([kernel, kernel], axis=0)
    bi = bias_in.reshape(1, 3 * UNITS)
    br = bias_rec.reshape(1, 3 * UNITS)
    rows_a = _sc_gather(table2, qrow_t[:_TH].reshape(_NW, _NCH, _CH))
    rows_b = _sc_gather(table2, qrow_t[_TH:].reshape(_NW, _NCH, _CH))
    xe_a = rows_a.reshape(_TH, BATCH, 128)
    xe_b = rows_b.reshape(_TH, BATCH, 128)
    out_a, st_a = _tc_gru_half(xe_a, par_t[:_TH], hidden, w2, rw=rec_kernel,
                               bi=bi, br=br, half=0)
    out, state = _tc_gru_half(xe_b, par_t[_TH:], st_a, w2, rw=rec_kernel,
                              bi=bi, br=br, half=1, obuf=out_a)
    return (jnp.swapaxes(out, 0, 1), state)


# speedup vs baseline: 1.0673x; 1.0004x over previous
import functools

import jax
import jax.numpy as jnp
from jax import lax
from jax.experimental import pallas as pl
from jax.experimental.pallas import tpu as pltpu
from jax.experimental.pallas import tpu_sc as plsc

VOCAB = 1000000
EMB = 64
UNITS = 128
BATCH = 1024
SEQ = 50

_VB = 32768
_QB = _VB // 4
_NPB = (VOCAB + _VB - 1) // _VB
_QROWS = _NPB * _QB

_NW = 32
_TH = SEQ // 2
_HROWS = BATCH * _TH
_RPW = _HROWS // _NW
_CH = 80
_NCH = _RPW // _CH

_BB = 512

_HI_MASK = -65536


def _pack_body(tT_ref, out_ref):
    x = lax.bitcast_convert_type(tT_ref[...], jnp.uint32)
    mask = jnp.uint32(0xffff0000)
    word_l = (x[:, :_QB] >> 16) | (x[:, 2 * _QB:3 * _QB] & mask)
    word_r = (x[:, _QB:2 * _QB] >> 16) | (x[:, 3 * _QB:] & mask)
    out_ref[:, :EMB] = lax.bitcast_convert_type(
        jnp.transpose(word_l), jnp.int32)
    out_ref[:, EMB:] = lax.bitcast_convert_type(
        jnp.transpose(word_r), jnp.int32)


def _tc_pack(tT):
    return pl.pallas_call(
        _pack_body,
        grid=(_NPB,),
        in_specs=[pl.BlockSpec((EMB, _VB), lambda i: (0, i))],
        out_specs=pl.BlockSpec((_QB, 128), lambda i: (i, 0)),
        out_shape=jax.ShapeDtypeStruct((_QROWS, 128), jnp.int32),
    )(tT)


def _sc_gather(table2, idx3):
    mesh = plsc.VectorSubcoreMesh(core_axis_name="c", subcore_axis_name="s")

    @functools.partial(
        pl.kernel,
        mesh=mesh,
        out_type=jax.ShapeDtypeStruct((_HROWS, 128), jnp.int32),
        scratch_types=[
            pltpu.VMEM((_NCH, _CH), jnp.int32),
            pltpu.VMEM((2, _CH, 128), jnp.int32),
            pltpu.SemaphoreType.DMA,
            pltpu.SemaphoreType.DMA,
        ],
    )
    def gather_kernel(table_hbm, idx_hbm, out_hbm, idx_v, buf_v, semg, semo):
        wid = lax.axis_index("s") * 2 + lax.axis_index("c")
        pltpu.sync_copy(idx_hbm.at[wid], idx_v)
        outs = []
        for j in range(_NCH):
            b = j % 2
            if j >= 2:
                outs[j - 2].wait()
            pltpu.async_copy(table_hbm.at[idx_v.at[j]], buf_v.at[b], semg).wait()
            outs.append(
                pltpu.async_copy(
                    buf_v.at[b],
                    out_hbm.at[pl.ds(wid * _RPW + j * _CH, _CH)],
                    semo,
                )
            )
        outs[-2].wait()
        outs[-1].wait()

    return gather_kernel(table2, idx3)


def _gru_body(xe_ref, par_ref, h0_ref, w2_ref, rw_ref, bi_ref, br_ref,
              out_ref, st_ref):
    h = h0_ref[...]
    w2 = w2_ref[...].astype(jnp.bfloat16)
    rw = rw_ref[...].astype(jnp.bfloat16)
    bi = bi_ref[...]
    br = br_ref[...]
    lane = lax.broadcasted_iota(jnp.int32, (_BB, 128), 1)
    hl = (lane >= EMB).astype(jnp.float32)
    for t in range(_TH):
        w = xe_ref[t]
        pf = par_ref[t].reshape(_BB, 1)
        s_sel = jnp.where(pf >= 2.0, 1.0, 0.0)
        h_sel = pf - 2.0 * s_sel
        shamt = (16.0 * (1.0 - s_sel)).astype(jnp.int32)
        bits = jnp.left_shift(w, shamt) & _HI_MASK
        xt = lax.bitcast_convert_type(bits, jnp.float32)
        hmatch = jnp.where(hl == h_sel, 1.0, 0.0)
        xt_m = (hmatch * xt).astype(jnp.bfloat16)
        gx = jnp.dot(xt_m, w2, preferred_element_type=jnp.float32) + bi
        gh = jnp.dot(h.astype(jnp.bfloat16), rw,
                     preferred_element_type=jnp.float32) + br
        xz = gx[:, :UNITS]
        xr = gx[:, UNITS:2 * UNITS]
        xh = gx[:, 2 * UNITS:]
        hz = gh[:, :UNITS]
        hr = gh[:, UNITS:2 * UNITS]
        hh = gh[:, 2 * UNITS:]
        z = jax.nn.sigmoid(xz + hz)
        r = jax.nn.sigmoid(xr + hr)
        hcand = jnp.tanh(xh + r * hh)
        h = z * h + (1.0 - z) * hcand
        out_ref[t] = h
    st_ref[...] = h


def _gru_body_aliased(xe_ref, par_ref, h0_ref, w2_ref, rw_ref, bi_ref, br_ref,
                      obuf_ref, out_ref, st_ref):
    del obuf_ref
    _gru_body(xe_ref, par_ref, h0_ref, w2_ref, rw_ref, bi_ref, br_ref,
              out_ref, st_ref)


def _tc_gru_half(xe, par, h0, w2, rw, bi, br, half, obuf=None):
    grid = (BATCH // _BB,)
    in_specs = [
        pl.BlockSpec((_TH, _BB, 128), lambda i: (0, i, 0)),
        pl.BlockSpec((_TH, _BB), lambda i: (0, i)),
        pl.BlockSpec((_BB, UNITS), lambda i: (i, 0)),
        pl.BlockSpec((128, 3 * UNITS), lambda i: (0, 0)),
        pl.BlockSpec((UNITS, 3 * UNITS), lambda i: (0, 0)),
        pl.BlockSpec((1, 3 * UNITS), lambda i: (0, 0)),
        pl.BlockSpec((1, 3 * UNITS), lambda i: (0, 0)),
    ]
    args = [xe, par, h0, w2, rw, bi, br]
    body = _gru_body
    aliases = {}
    if obuf is not None:
        in_specs.append(pl.BlockSpec(memory_space=pl.ANY))
        args.append(obuf)
        body = _gru_body_aliased
        aliases = {7: 0}
    out, state = pl.pallas_call(
        body,
        grid=grid,
        in_specs=in_specs,
        out_specs=[
            pl.BlockSpec((_TH, _BB, UNITS), lambda i, h=half: (h, i, 0)),
            pl.BlockSpec((_BB, UNITS), lambda i: (i, 0)),
        ],
        out_shape=[
            jax.ShapeDtypeStruct((SEQ, BATCH, UNITS), jnp.float32),
            jax.ShapeDtypeStruct((BATCH, UNITS), jnp.float32),
        ],
        input_output_aliases=aliases,
    )(*args)
    return out, state


def kernel(x, hidden, emb_table, kernel, rec_kernel, bias_in, bias_rec):
    xi = x.astype(jnp.int32)
    blk = xi // _VB
    r = xi % _VB
    quarter = r // _QB
    qrow = blk * _QB + (r % _QB)
    qrow_t = jnp.transpose(qrow)
    par_t = jnp.transpose(quarter).astype(jnp.float32)
    table2 = _tc_pack(jnp.transpose(emb_table))
    w2 = jnp.concatenate([kernel, kernel], axis=0)
    bi = bias_in.reshape(1, 3 * UNITS)
    br = bias_rec.reshape(1, 3 * UNITS)
    rows_a = _sc_gather(table2, qrow_t[:_TH].reshape(_NW, _NCH, _CH))
    rows_b = _sc_gather(table2, qrow_t[_TH:].reshape(_NW, _NCH, _CH))
    xe_a = rows_a.reshape(_TH, BATCH, 128)
    xe_b = rows_b.reshape(_TH, BATCH, 128)
    out_a, st_a = _tc_gru_half(xe_a, par_t[:_TH], hidden, w2, rw=rec_kernel,
                               bi=bi, br=br, half=0)
    out, state = _tc_gru_half(xe_b, par_t[_TH:], st_a, w2, rw=rec_kernel,
                              bi=bi, br=br, half=1, obuf=out_a)
    return (jnp.swapaxes(out, 0, 1), state)


# speedup vs baseline: 1.0949x; 1.0258x over previous
import functools

import jax
import jax.numpy as jnp
from jax import lax
from jax.experimental import pallas as pl
from jax.experimental.pallas import tpu as pltpu
from jax.experimental.pallas import tpu_sc as plsc

VOCAB = 1000000
EMB = 64
UNITS = 128
BATCH = 1024
SEQ = 50

_VB = 32768
_QB = _VB // 4
_NPB = (VOCAB + _VB - 1) // _VB
_QROWS = _NPB * _QB

_NW = 32
_TH = SEQ // 2
_HROWS = BATCH * _TH
_RPW = _HROWS // _NW
_CH = 80
_NCH = _RPW // _CH

_BB = 512

_HI_MASK = -65536


def _pack_body(tT_ref, out_ref):
    x = lax.bitcast_convert_type(tT_ref[...], jnp.uint32)
    mask = jnp.uint32(0xffff0000)
    word_l = (x[:, :_QB] >> 16) | (x[:, 2 * _QB:3 * _QB] & mask)
    word_r = (x[:, _QB:2 * _QB] >> 16) | (x[:, 3 * _QB:] & mask)
    out_ref[:, :EMB] = lax.bitcast_convert_type(
        jnp.transpose(word_l), jnp.int32)
    out_ref[:, EMB:] = lax.bitcast_convert_type(
        jnp.transpose(word_r), jnp.int32)


def _tc_pack(tT):
    return pl.pallas_call(
        _pack_body,
        grid=(_NPB,),
        in_specs=[pl.BlockSpec((EMB, _VB), lambda i: (0, i))],
        out_specs=pl.BlockSpec((_QB, 128), lambda i: (i, 0)),
        out_shape=jax.ShapeDtypeStruct((_QROWS, 128), jnp.int32),
    )(tT)


def _sc_gather(table2, idx3):
    mesh = plsc.VectorSubcoreMesh(core_axis_name="c", subcore_axis_name="s")

    @functools.partial(
        pl.kernel,
        mesh=mesh,
        out_type=jax.ShapeDtypeStruct((_HROWS, 128), jnp.int32),
        scratch_types=[
            pltpu.VMEM((_NCH, _CH), jnp.int32),
            pltpu.VMEM((4, _CH, 128), jnp.int32),
            pltpu.SemaphoreType.DMA,
            pltpu.SemaphoreType.DMA,
            pltpu.SemaphoreType.DMA,
            pltpu.SemaphoreType.DMA,
            pltpu.SemaphoreType.DMA,
        ],
    )
    def gather_kernel(table_hbm, idx_hbm, out_hbm, idx_v, buf_v,
                      sg0, sg1, sg2, sg3, semo):
        wid = lax.axis_index("s") * 2 + lax.axis_index("c")
        pltpu.sync_copy(idx_hbm.at[wid], idx_v)
        semg = [sg0, sg1, sg2, sg3]
        gath = []
        outs = []
        for j in range(_NCH):
            b = j % 4
            if j >= 4:
                outs[j - 4].wait()
            gath.append(
                pltpu.async_copy(table_hbm.at[idx_v.at[j]], buf_v.at[b],
                                 semg[b])
            )
            if j >= 1:
                gath[j - 1].wait()
                outs.append(
                    pltpu.async_copy(
                        buf_v.at[(j - 1) % 4],
                        out_hbm.at[pl.ds(wid * _RPW + (j - 1) * _CH, _CH)],
                        semo,
                    )
                )
        gath[-1].wait()
        outs.append(
            pltpu.async_copy(
                buf_v.at[(_NCH - 1) % 4],
                out_hbm.at[pl.ds(wid * _RPW + (_NCH - 1) * _CH, _CH)],
                semo,
            )
        )
        for o in outs[-4:]:
            o.wait()

    return gather_kernel(table2, idx3)


def _gru_body(xe_ref, par_ref, h0_ref, w2_ref, rw_ref, bi_ref, br_ref,
              out_ref, st_ref):
    h = h0_ref[...]
    w2 = w2_ref[...].astype(jnp.bfloat16)
    rw = rw_ref[...].astype(jnp.bfloat16)
    bi = bi_ref[...]
    br = br_ref[...]
    lane = lax.broadcasted_iota(jnp.int32, (_BB, 128), 1)
    hl = (lane >= EMB).astype(jnp.float32)
    for t in range(_TH):
        w = xe_ref[t]
        pf = par_ref[t].reshape(_BB, 1)
        s_sel = jnp.where(pf >= 2.0, 1.0, 0.0)
        h_sel = pf - 2.0 * s_sel
        shamt = (16.0 * (1.0 - s_sel)).astype(jnp.int32)
        bits = jnp.left_shift(w, shamt) & _HI_MASK
        xt = lax.bitcast_convert_type(bits, jnp.float32)
        hmatch = jnp.where(hl == h_sel, 1.0, 0.0)
        xt_m = (hmatch * xt).astype(jnp.bfloat16)
        gx = jnp.dot(xt_m, w2, preferred_element_type=jnp.float32) + bi
        gh = jnp.dot(h.astype(jnp.bfloat16), rw,
                     preferred_element_type=jnp.float32) + br
        xz = gx[:, :UNITS]
        xr = gx[:, UNITS:2 * UNITS]
        xh = gx[:, 2 * UNITS:]
        hz = gh[:, :UNITS]
        hr = gh[:, UNITS:2 * UNITS]
        hh = gh[:, 2 * UNITS:]
        z = jax.nn.sigmoid(xz + hz)
        r = jax.nn.sigmoid(xr + hr)
        hcand = jnp.tanh(xh + r * hh)
        h = z * h + (1.0 - z) * hcand
        out_ref[t] = h
    st_ref[...] = h


def _gru_body_aliased(xe_ref, par_ref, h0_ref, w2_ref, rw_ref, bi_ref, br_ref,
                      obuf_ref, out_ref, st_ref):
    del obuf_ref
    _gru_body(xe_ref, par_ref, h0_ref, w2_ref, rw_ref, bi_ref, br_ref,
              out_ref, st_ref)


def _tc_gru_half(xe, par, h0, w2, rw, bi, br, half, obuf=None):
    grid = (BATCH // _BB,)
    in_specs = [
        pl.BlockSpec((_TH, _BB, 128), lambda i: (0, i, 0)),
        pl.BlockSpec((_TH, _BB), lambda i: (0, i)),
        pl.BlockSpec((_BB, UNITS), lambda i: (i, 0)),
        pl.BlockSpec((128, 3 * UNITS), lambda i: (0, 0)),
        pl.BlockSpec((UNITS, 3 * UNITS), lambda i: (0, 0)),
        pl.BlockSpec((1, 3 * UNITS), lambda i: (0, 0)),
        pl.BlockSpec((1, 3 * UNITS), lambda i: (0, 0)),
    ]
    args = [xe, par, h0, w2, rw, bi, br]
    body = _gru_body
    aliases = {}
    if obuf is not None:
        in_specs.append(pl.BlockSpec(memory_space=pl.ANY))
        args.append(obuf)
        body = _gru_body_aliased
        aliases = {7: 0}
    out, state = pl.pallas_call(
        body,
        grid=grid,
        in_specs=in_specs,
        out_specs=[
            pl.BlockSpec((_TH, _BB, UNITS), lambda i, h=half: (h, i, 0)),
            pl.BlockSpec((_BB, UNITS), lambda i: (i, 0)),
        ],
        out_shape=[
            jax.ShapeDtypeStruct((SEQ, BATCH, UNITS), jnp.float32),
            jax.ShapeDtypeStruct((BATCH, UNITS), jnp.float32),
        ],
        input_output_aliases=aliases,
    )(*args)
    return out, state


def kernel(x, hidden, emb_table, kernel, rec_kernel, bias_in, bias_rec):
    xi = x.astype(jnp.int32)
    blk = xi // _VB
    r = xi % _VB
    quarter = r // _QB
    qrow = blk * _QB + (r % _QB)
    qrow_t = jnp.transpose(qrow)
    par_t = jnp.transpose(quarter).astype(jnp.float32)
    table2 = _tc_pack(jnp.transpose(emb_table))
    w2 = jnp.concatenate([kernel, kernel], axis=0)
    bi = bias_in.reshape(1, 3 * UNITS)
    br = bias_rec.reshape(1, 3 * UNITS)
    rows_a = _sc_gather(table2, qrow_t[:_TH].reshape(_NW, _NCH, _CH))
    rows_b = _sc_gather(table2, qrow_t[_TH:].reshape(_NW, _NCH, _CH))
    xe_a = rows_a.reshape(_TH, BATCH, 128)
    xe_b = rows_b.reshape(_TH, BATCH, 128)
    out_a, st_a = _tc_gru_half(xe_a, par_t[:_TH], hidden, w2, rw=rec_kernel,
                               bi=bi, br=br, half=0)
    out, state = _tc_gru_half(xe_b, par_t[_TH:], st_a, w2, rw=rec_kernel,
                              bi=bi, br=br, half=1, obuf=out_a)
    return (jnp.swapaxes(out, 0, 1), state)


# speedup vs baseline: 1.1018x; 1.0063x over previous
import functools

import jax
import jax.numpy as jnp
from jax import lax
from jax.experimental import pallas as pl
from jax.experimental.pallas import tpu as pltpu
from jax.experimental.pallas import tpu_sc as plsc

VOCAB = 1000000
EMB = 64
UNITS = 128
BATCH = 1024
SEQ = 50

_VB = 32768
_QB = _VB // 4
_NPB = (VOCAB + _VB - 1) // _VB
_QROWS = _NPB * _QB

_NW = 32
_TH = SEQ // 2
_HROWS = BATCH * _TH
_RPW = _HROWS // _NW
_CH = 80
_NCH = _RPW // _CH

_BB = 512

_HI_MASK = -65536


def _pack_body(tT_ref, out_ref):
    x = lax.bitcast_convert_type(tT_ref[...], jnp.uint32)
    mask = jnp.uint32(0xffff0000)
    word_l = (x[:, :_QB] >> 16) | (x[:, 2 * _QB:3 * _QB] & mask)
    word_r = (x[:, _QB:2 * _QB] >> 16) | (x[:, 3 * _QB:] & mask)
    out_ref[:, :EMB] = lax.bitcast_convert_type(
        jnp.transpose(word_l), jnp.int32)
    out_ref[:, EMB:] = lax.bitcast_convert_type(
        jnp.transpose(word_r), jnp.int32)


def _tc_pack(tT):
    return pl.pallas_call(
        _pack_body,
        grid=(_NPB,),
        in_specs=[pl.BlockSpec((EMB, _VB), lambda i: (0, i))],
        out_specs=pl.BlockSpec((_QB, 128), lambda i: (i, 0)),
        out_shape=jax.ShapeDtypeStruct((_QROWS, 128), jnp.int32),
    )(tT)


def _sc_gather(table2, idx3):
    mesh = plsc.VectorSubcoreMesh(core_axis_name="c", subcore_axis_name="s")

    @functools.partial(
        pl.kernel,
        mesh=mesh,
        out_type=jax.ShapeDtypeStruct((_HROWS, 128), jnp.int32),
        scratch_types=[
            pltpu.VMEM((_NCH, _CH), jnp.int32),
            pltpu.VMEM((4, _CH, 128), jnp.int32),
            pltpu.SemaphoreType.DMA,
            pltpu.SemaphoreType.DMA,
            pltpu.SemaphoreType.DMA,
            pltpu.SemaphoreType.DMA,
            pltpu.SemaphoreType.DMA,
        ],
    )
    def gather_kernel(table_hbm, idx_hbm, out_hbm, idx_v, buf_v,
                      sg0, sg1, sg2, sg3, semo):
        wid = lax.axis_index("s") * 2 + lax.axis_index("c")
        pltpu.sync_copy(idx_hbm.at[wid], idx_v)
        semg = [sg0, sg1, sg2, sg3]
        gath = []
        outs = []
        for j in range(_NCH):
            b = j % 4
            if j >= 4:
                outs[j - 4].wait()
            gath.append(
                pltpu.async_copy(table_hbm.at[idx_v.at[j]], buf_v.at[b],
                                 semg[b])
            )
            if j >= 2:
                gath[j - 2].wait()
                outs.append(
                    pltpu.async_copy(
                        buf_v.at[(j - 2) % 4],
                        out_hbm.at[pl.ds(wid * _RPW + (j - 2) * _CH, _CH)],
                        semo,
                    )
                )
        for j in (_NCH - 2, _NCH - 1):
            gath[j].wait()
            outs.append(
                pltpu.async_copy(
                    buf_v.at[j % 4],
                    out_hbm.at[pl.ds(wid * _RPW + j * _CH, _CH)],
                    semo,
                )
            )
        for o in outs[-4:]:
            o.wait()

    return gather_kernel(table2, idx3)


def _gru_body(xe_ref, par_ref, h0_ref, w2_ref, rw_ref, bi_ref, br_ref,
              out_ref, st_ref):
    h = h0_ref[...]
    w2 = w2_ref[...].astype(jnp.bfloat16)
    rw = rw_ref[...].astype(jnp.bfloat16)
    bi = bi_ref[...]
    br = br_ref[...]
    lane = lax.broadcasted_iota(jnp.int32, (_BB, 128), 1)
    hl = (lane >= EMB).astype(jnp.float32)
    for t in range(_TH):
        w = xe_ref[t]
        pf = par_ref[t].reshape(_BB, 1)
        s_sel = jnp.where(pf >= 2.0, 1.0, 0.0)
        h_sel = pf - 2.0 * s_sel
        shamt = (16.0 * (1.0 - s_sel)).astype(jnp.int32)
        bits = jnp.left_shift(w, shamt) & _HI_MASK
        xt = lax.bitcast_convert_type(bits, jnp.float32)
        hmatch = jnp.where(hl == h_sel, 1.0, 0.0)
        xt_m = (hmatch * xt).astype(jnp.bfloat16)
        gx = jnp.dot(xt_m, w2, preferred_element_type=jnp.float32) + bi
        gh = jnp.dot(h.astype(jnp.bfloat16), rw,
                     preferred_element_type=jnp.float32) + br
        xz = gx[:, :UNITS]
        xr = gx[:, UNITS:2 * UNITS]
        xh = gx[:, 2 * UNITS:]
        hz = gh[:, :UNITS]
        hr = gh[:, UNITS:2 * UNITS]
        hh = gh[:, 2 * UNITS:]
        z = jax.nn.sigmoid(xz + hz)
        r = jax.nn.sigmoid(xr + hr)
        hcand = jnp.tanh(xh + r * hh)
        h = z * h + (1.0 - z) * hcand
        out_ref[t] = h
    st_ref[...] = h


def _gru_body_aliased(xe_ref, par_ref, h0_ref, w2_ref, rw_ref, bi_ref, br_ref,
                      obuf_ref, out_ref, st_ref):
    del obuf_ref
    _gru_body(xe_ref, par_ref, h0_ref, w2_ref, rw_ref, bi_ref, br_ref,
              out_ref, st_ref)


def _tc_gru_half(xe, par, h0, w2, rw, bi, br, half, obuf=None):
    grid = (BATCH // _BB,)
    in_specs = [
        pl.BlockSpec((_TH, _BB, 128), lambda i: (0, i, 0)),
        pl.BlockSpec((_TH, _BB), lambda i: (0, i)),
        pl.BlockSpec((_BB, UNITS), lambda i: (i, 0)),
        pl.BlockSpec((128, 3 * UNITS), lambda i: (0, 0)),
        pl.BlockSpec((UNITS, 3 * UNITS), lambda i: (0, 0)),
        pl.BlockSpec((1, 3 * UNITS), lambda i: (0, 0)),
        pl.BlockSpec((1, 3 * UNITS), lambda i: (0, 0)),
    ]
    args = [xe, par, h0, w2, rw, bi, br]
    body = _gru_body
    aliases = {}
    if obuf is not None:
        in_specs.append(pl.BlockSpec(memory_space=pl.ANY))
        args.append(obuf)
        body = _gru_body_aliased
        aliases = {7: 0}
    out, state = pl.pallas_call(
        body,
        grid=grid,
        in_specs=in_specs,
        out_specs=[
            pl.BlockSpec((_TH, _BB, UNITS), lambda i, h=half: (h, i, 0)),
            pl.BlockSpec((_BB, UNITS), lambda i: (i, 0)),
        ],
        out_shape=[
            jax.ShapeDtypeStruct((SEQ, BATCH, UNITS), jnp.float32),
            jax.ShapeDtypeStruct((BATCH, UNITS), jnp.float32),
        ],
        input_output_aliases=aliases,
    )(*args)
    return out, state


def kernel(x, hidden, emb_table, kernel, rec_kernel, bias_in, bias_rec):
    xi = x.astype(jnp.int32)
    blk = xi // _VB
    r = xi % _VB
    quarter = r // _QB
    qrow = blk * _QB + (r % _QB)
    qrow_t = jnp.transpose(qrow)
    par_t = jnp.transpose(quarter).astype(jnp.float32)
    table2 = _tc_pack(jnp.transpose(emb_table))
    w2 = jnp.concatenate([kernel, kernel], axis=0)
    bi = bias_in.reshape(1, 3 * UNITS)
    br = bias_rec.reshape(1, 3 * UNITS)
    rows_a = _sc_gather(table2, qrow_t[:_TH].reshape(_NW, _NCH, _CH))
    rows_b = _sc_gather(table2, qrow_t[_TH:].reshape(_NW, _NCH, _CH))
    xe_a = rows_a.reshape(_TH, BATCH, 128)
    xe_b = rows_b.reshape(_TH, BATCH, 128)
    out_a, st_a = _tc_gru_half(xe_a, par_t[:_TH], hidden, w2, rw=rec_kernel,
                               bi=bi, br=br, half=0)
    out, state = _tc_gru_half(xe_b, par_t[_TH:], st_a, w2, rw=rec_kernel,
                              bi=bi, br=br, half=1, obuf=out_a)
    return (jnp.swapaxes(out, 0, 1), state)
